# Initial kernel scaffold; baseline (speedup 1.0000x reference)
import jax
import jax.numpy as jnp
from jax.experimental import pallas as pl


def kernel(features, edge_index, W1, W2, Wc):
    raise NotImplementedError("write your pallas kernel here")


# speedup vs baseline: 6.1513x; 6.1513x over previous
import functools

import jax
import jax.numpy as jnp
from jax import lax
from jax.experimental import pallas as pl
from jax.experimental.pallas import tpu as pltpu
from jax.experimental.pallas import tpu_sc as plsc

N_NODES = 10000
N_EDGES = 320000
D_IN = 128
D_HID = 128
D_HALF = 64
D_OUT = 16

NC = 2
NS = 16
NW = NC * NS
CHUNK = 128
NCHUNK = N_EDGES // CHUNK
N_PAD = 10240
ROWS_PER_SUB = N_PAD // NS
DEG_W = 16

_MESH = plsc.VectorSubcoreMesh(core_axis_name="c", subcore_axis_name="s")
_SC_PARAMS = pltpu.CompilerParams(use_tc_tiling_on_sc=False)


def _fill(ref, n_rows, width, value):
    per_row = width // 16

    def body(i, carry):
        ref[i // per_row, pl.ds((i % per_row) * 16, 16)] = jnp.full(
            (16,), value, jnp.float32)
        return carry

    lax.fori_loop(0, n_rows * per_row, body, 0)


def _chunk_range(wid):
    base = NCHUNK // NW
    rem = NCHUNK % NW
    start = wid * base + jnp.minimum(wid, rem)
    count = base + (wid < rem).astype(jnp.int32)
    return start, count


@functools.partial(
    pl.kernel,
    out_type=(
        jax.ShapeDtypeStruct((NC, N_PAD, DEG_W), jnp.float32),
        jax.ShapeDtypeStruct((NC, N_PAD, DEG_W), jnp.float32),
    ),
    mesh=_MESH,
    compiler_params=_SC_PARAMS,
    scratch_types=[
        pltpu.VMEM((CHUNK,), jnp.int32),
        pltpu.VMEM((CHUNK,), jnp.int32),
        pltpu.VMEM((CHUNK, DEG_W), jnp.float32),
        pltpu.VMEM((ROWS_PER_SUB, DEG_W), jnp.float32),
        pltpu.VMEM_SHARED((N_PAD, DEG_W), jnp.float32),
        pltpu.VMEM_SHARED((N_PAD, DEG_W), jnp.float32),
    ],
)
def _deg_kernel(src_e, dst_e, out_s, out_d, idx_s, idx_d, ones_v, zero_v,
                acc_s, acc_d):
    c = lax.axis_index("c")
    s = lax.axis_index("s")
    wid = c * NS + s
    _fill(ones_v, CHUNK, DEG_W, 1.0)
    _fill(zero_v, ROWS_PER_SUB, DEG_W, 0.0)
    base = s * ROWS_PER_SUB
    pltpu.sync_copy(zero_v, acc_s.at[pl.ds(base, ROWS_PER_SUB)])
    pltpu.sync_copy(zero_v, acc_d.at[pl.ds(base, ROWS_PER_SUB)])
    plsc.subcore_barrier()

    start, count = _chunk_range(wid)

    def eb(j, carry):
        eo = (start + j) * CHUNK
        pltpu.sync_copy(src_e.at[pl.ds(eo, CHUNK)], idx_s)
        pltpu.sync_copy(dst_e.at[pl.ds(eo, CHUNK)], idx_d)
        pltpu.sync_copy(ones_v, acc_s.at[idx_s], add=True)
        pltpu.sync_copy(ones_v, acc_d.at[idx_d], add=True)
        return carry

    lax.fori_loop(0, count, eb, 0)
    plsc.subcore_barrier()
    pltpu.sync_copy(acc_s.at[pl.ds(base, ROWS_PER_SUB)],
                    out_s.at[c, pl.ds(base, ROWS_PER_SUB)])
    pltpu.sync_copy(acc_d.at[pl.ds(base, ROWS_PER_SUB)],
                    out_d.at[c, pl.ds(base, ROWS_PER_SUB)])


def _make_edge_pass(d):
    @functools.partial(
        pl.kernel,
        out_type=jax.ShapeDtypeStruct((NC, N_PAD, d), jnp.float32),
        mesh=_MESH,
        compiler_params=_SC_PARAMS,
        scratch_types=[
            pltpu.VMEM((CHUNK,), jnp.int32),
            pltpu.VMEM((CHUNK,), jnp.int32),
            pltpu.VMEM((CHUNK, d), jnp.float32),
            pltpu.VMEM_SHARED((N_PAD, d), jnp.float32),
            pltpu.SemaphoreType.DMA,
        ],
    )
    def k(h_hbm, src_e, dst_e, out_hbm, idx_s, idx_d, rows, acc, sem):
        c = lax.axis_index("c")
        s = lax.axis_index("s")
        wid = c * NS + s
        _fill(rows, CHUNK, d, 0.0)
        base = s * ROWS_PER_SUB
        off = 0
        while off < ROWS_PER_SUB:
            sz = min(CHUNK, ROWS_PER_SUB - off)
            pltpu.sync_copy(rows.at[pl.ds(0, sz)],
                            acc.at[pl.ds(base + off, sz)])
            off += sz
        plsc.subcore_barrier()

        start, count = _chunk_range(wid)

        def eb(j, carry):
            eo = (start + j) * CHUNK
            pltpu.sync_copy(src_e.at[pl.ds(eo, CHUNK)], idx_s)
            pltpu.sync_copy(dst_e.at[pl.ds(eo, CHUNK)], idx_d)
            pltpu.async_copy(h_hbm.at[idx_s], rows, sem).wait()
            pltpu.sync_copy(rows, acc.at[idx_d], add=True)
            return carry

        lax.fori_loop(0, count, eb, 0)
        plsc.subcore_barrier()
        pltpu.sync_copy(acc.at[pl.ds(base, ROWS_PER_SUB)],
                        out_hbm.at[c, pl.ds(base, ROWS_PER_SUB)])

    return k


_edge_pass_128 = _make_edge_pass(D_HID)
_edge_pass_64 = _make_edge_pass(D_HALF)


def _scale_body(x_ref, d0, d1, o_ref):
    norm = lax.rsqrt(jnp.maximum(d0[...] + d1[...], 1.0))
    o_ref[...] = x_ref[...] * norm


def _mid_body(p0, p1, dd0, dd1, sd0, sd1, w1, w2, o_ref):
    nd = lax.rsqrt(jnp.maximum(dd0[...] + dd1[...], 1.0))
    agg = (p0[...] + p1[...]) * nd
    h1 = jnp.dot(agg, w1[...], preferred_element_type=jnp.float32)
    h1 = jnp.where(h1 >= 0.0, h1, 0.01 * h1)
    ns = lax.rsqrt(jnp.maximum(sd0[...] + sd1[...], 1.0))
    o_ref[...] = jnp.dot(h1, w2[...], preferred_element_type=jnp.float32) * ns


def _fin_body(q0, q1, dd0, dd1, wc, o_ref):
    nd = lax.rsqrt(jnp.maximum(dd0[...] + dd1[...], 1.0))
    h2 = (q0[...] + q1[...]) * nd
    h2 = jnp.where(h2 >= 0.0, h2, 0.01 * h2)
    pooled = jnp.sum(h2, axis=0, keepdims=True) * (1.0 / N_NODES)
    o_ref[...] = jnp.dot(pooled, wc[...], preferred_element_type=jnp.float32)


def kernel(features, edge_index, W1, W2, Wc):
    src = edge_index[0].astype(jnp.int32)
    dst = edge_index[1].astype(jnp.int32)

    deg_s, deg_d = _deg_kernel(src, dst)
    ds0, ds1 = deg_s[0, :N_NODES, 0:1], deg_s[1, :N_NODES, 0:1]
    dd0, dd1 = deg_d[0, :N_NODES, 0:1], deg_d[1, :N_NODES, 0:1]

    h = pl.pallas_call(
        _scale_body,
        out_shape=jax.ShapeDtypeStruct((N_NODES, D_IN), jnp.float32),
    )(features, ds0, ds1)

    part1 = _edge_pass_128(h, src, dst)

    g = pl.pallas_call(
        _mid_body,
        out_shape=jax.ShapeDtypeStruct((N_NODES, D_HALF), jnp.float32),
    )(part1[0, :N_NODES], part1[1, :N_NODES], dd0, dd1, ds0, ds1, W1, W2)

    part2 = _edge_pass_64(g, src, dst)

    out = pl.pallas_call(
        _fin_body,
        out_shape=jax.ShapeDtypeStruct((1, D_OUT), jnp.float32),
    )(part2[0, :N_NODES], part2[1, :N_NODES], dd0, dd1, Wc)
    return out
